# Initial kernel scaffold; baseline (speedup 1.0000x reference)
import functools

import jax
import jax.numpy as jnp
from jax import lax
from jax.experimental import pallas as pl
from jax.experimental.pallas import tpu as pltpu
from jax.experimental.pallas import tpu_sc as plsc

DIM = 64
BATCH = 16384
NC = 2
NS = 16
NW = NC * NS
ROWS_PER_W = BATCH // NW
L = 16
IDX_MINOR = 128
IDX_ROWS = ROWS_PER_W // IDX_MINOR
GROUPS = ROWS_PER_W // L


def _sc_body(sub_hbm, obj_hbm, rela_hbm, diag_hbm, out_hbm,
             idx_v, sub_v, obj_v, rel_v, t_v, out_v, sem, gsem):
    wid = lax.axis_index("s") * NC + lax.axis_index("c")
    base = wid * ROWS_PER_W

    pltpu.sync_copy(rela_hbm.at[pl.ds(wid * IDX_ROWS, IDX_ROWS)], idx_v)

    gathers = [
        pltpu.async_copy(
            diag_hbm.at[idx_v.at[j]],
            rel_v.at[pl.ds(j * IDX_MINOR, IDX_MINOR)],
            gsem,
        )
        for j in range(IDX_ROWS)
    ]
    sub_cp = pltpu.async_copy(sub_hbm.at[pl.ds(base, ROWS_PER_W)], sub_v, sem)
    obj_cp = pltpu.async_copy(obj_hbm.at[pl.ds(base, ROWS_PER_W)], obj_v, sem)
    for cp in gathers:
        cp.wait()
    sub_cp.wait()
    obj_cp.wait()

    col_idx = lax.iota(jnp.int32, L)

    def group(g, carry):
        for j in range(L):
            row = g * L + j
            acc = None
            for c in range(DIM // L):
                s = sub_v[row, pl.ds(c * L, L)]
                r = rel_v[row, pl.ds(c * L, L)]
                o = obj_v[row, pl.ds(c * L, L)]
                p = s * r * o
                acc = p if acc is None else acc + p
            t_v[j, pl.ds(0, L)] = acc
        tot = plsc.load_gather(t_v, [col_idx, jnp.full((L,), 0, jnp.int32)])
        for d in range(1, L):
            tot = tot + plsc.load_gather(
                t_v, [col_idx, jnp.full((L,), d, jnp.int32)])
        out_v[pl.ds(g * L, L)] = tot
        return carry

    lax.fori_loop(0, GROUPS, group, 0)

    pltpu.sync_copy(out_v, out_hbm.at[pl.ds(base, ROWS_PER_W)])


@functools.partial(
    pl.kernel,
    out_type=jax.ShapeDtypeStruct((BATCH,), jnp.float32),
    mesh=plsc.VectorSubcoreMesh(core_axis_name="c", subcore_axis_name="s"),
    scratch_types=[
        pltpu.VMEM((IDX_ROWS, IDX_MINOR), jnp.int32),
        pltpu.VMEM((ROWS_PER_W, DIM), jnp.float32),
        pltpu.VMEM((ROWS_PER_W, DIM), jnp.float32),
        pltpu.VMEM((ROWS_PER_W, DIM), jnp.float32),
        pltpu.VMEM((L, L + 1), jnp.float32),
        pltpu.VMEM((ROWS_PER_W,), jnp.float32),
        pltpu.SemaphoreType.DMA,
        pltpu.SemaphoreType.DMA,
    ],
)
def _dist_mult_sc(sub_hbm, obj_hbm, rela_hbm, diag_hbm, out_hbm, *scratch):
    _sc_body(sub_hbm, obj_hbm, rela_hbm, diag_hbm, out_hbm, *scratch)


def kernel(sub_embed, obj_embed, rela, diag):
    rela2d = rela.astype(jnp.int32).reshape(NW * IDX_ROWS, IDX_MINOR)
    return _dist_mult_sc(sub_embed, obj_embed, rela2d, diag)


# speedup vs baseline: 1.0489x; 1.0489x over previous
import functools

import jax
import jax.numpy as jnp
from jax import lax
from jax.experimental import pallas as pl
from jax.experimental.pallas import tpu as pltpu
from jax.experimental.pallas import tpu_sc as plsc

DIM = 64
BATCH = 16384
NC = 2
NS = 16
NW = NC * NS
ROWS_PER_W = BATCH // NW
L = 16
IDX_MINOR = 128
IDX_ROWS = ROWS_PER_W // IDX_MINOR
GROUPS = ROWS_PER_W // L


def _sc_body(sub_hbm, obj_hbm, rela_hbm, diag_hbm, out_hbm,
             idx_v, sub_v, obj_v, rel_v, out_v, sem, gsem):
    wid = lax.axis_index("s") * NC + lax.axis_index("c")
    base = wid * ROWS_PER_W

    pltpu.sync_copy(rela_hbm.at[pl.ds(wid * IDX_ROWS, IDX_ROWS)], idx_v)

    gathers = [
        pltpu.async_copy(
            diag_hbm.at[idx_v.at[j]],
            rel_v.at[pl.ds(j * IDX_MINOR, IDX_MINOR)],
            gsem,
        )
        for j in range(IDX_ROWS)
    ]
    sub_cp = pltpu.async_copy(sub_hbm.at[pl.ds(base, ROWS_PER_W)], sub_v, sem)
    obj_cp = pltpu.async_copy(obj_hbm.at[pl.ds(base, ROWS_PER_W)], obj_v, sem)
    for cp in gathers:
        cp.wait()
    sub_cp.wait()
    obj_cp.wait()

    col_idx = lax.iota(jnp.int32, L)

    def group(g, carry):
        tot = jnp.zeros((L,), jnp.float32)
        for j in range(L):
            row = g * L + j
            acc = None
            for c in range(DIM // L):
                s = sub_v[row, pl.ds(c * L, L)]
                r = rel_v[row, pl.ds(c * L, L)]
                o = obj_v[row, pl.ds(c * L, L)]
                p = s * r * o
                acc = p if acc is None else acc + p
            tot = jnp.where(col_idx == j, jnp.sum(acc), tot)
        out_v[pl.ds(g * L, L)] = tot
        return carry

    lax.fori_loop(0, GROUPS, group, 0)

    pltpu.sync_copy(out_v, out_hbm.at[pl.ds(base, ROWS_PER_W)])


@functools.partial(
    pl.kernel,
    out_type=jax.ShapeDtypeStruct((BATCH,), jnp.float32),
    mesh=plsc.VectorSubcoreMesh(core_axis_name="c", subcore_axis_name="s"),
    compiler_params=pltpu.CompilerParams(
        needs_layout_passes=False, use_tc_tiling_on_sc=False),
    scratch_types=[
        pltpu.VMEM((IDX_ROWS, IDX_MINOR), jnp.int32),
        pltpu.VMEM((ROWS_PER_W, DIM), jnp.float32),
        pltpu.VMEM((ROWS_PER_W, DIM), jnp.float32),
        pltpu.VMEM((ROWS_PER_W, DIM), jnp.float32),
        pltpu.VMEM((ROWS_PER_W,), jnp.float32),
        pltpu.SemaphoreType.DMA,
        pltpu.SemaphoreType.DMA,
    ],
)
def _dist_mult_sc(sub_hbm, obj_hbm, rela_hbm, diag_hbm, out_hbm, *scratch):
    _sc_body(sub_hbm, obj_hbm, rela_hbm, diag_hbm, out_hbm, *scratch)


def kernel(sub_embed, obj_embed, rela, diag):
    rela2d = rela.astype(jnp.int32).reshape(NW * IDX_ROWS, IDX_MINOR)
    return _dist_mult_sc(sub_embed, obj_embed, rela2d, diag)


# speedup vs baseline: 1.2815x; 1.2218x over previous
import functools

import jax
import jax.numpy as jnp
from jax import lax
from jax.experimental import pallas as pl
from jax.experimental.pallas import tpu as pltpu
from jax.experimental.pallas import tpu_sc as plsc

DIM = 64
PAD_DIM = 128
BATCH = 16384
NC = 2
NS = 16
NW = NC * NS
ROWS_PER_W = BATCH // NW
L = 16
CH = 128
N_CHUNKS = ROWS_PER_W // CH
CH_GROUPS = CH // L


def _sc_body(sub_hbm, obj_hbm, rela_hbm, diag_hbm, out_hbm,
             idx_v, sub_v0, obj_v0, rel_v0, sub_v1, obj_v1, rel_v1,
             out_v, sem0, sem1):
    wid = lax.axis_index("s") * NC + lax.axis_index("c")
    base = wid * ROWS_PER_W

    bufs = ((sub_v0, obj_v0, rel_v0, sem0), (sub_v1, obj_v1, rel_v1, sem1))

    pltpu.sync_copy(rela_hbm.at[pl.ds(base, ROWS_PER_W)], idx_v)

    def fire(k):
        sub_vb, obj_vb, rel_vb, semb = bufs[k % 2]
        cbase = base + k * CH
        return (
            pltpu.async_copy(
                diag_hbm.at[idx_v.at[pl.ds(k * CH, CH)]], rel_vb, semb),
            pltpu.async_copy(sub_hbm.at[pl.ds(cbase, CH)], sub_vb, semb),
            pltpu.async_copy(obj_hbm.at[pl.ds(cbase, CH)], obj_vb, semb),
        )

    col_idx = lax.iota(jnp.int32, L)

    def compute(k):
        sub_vb, obj_vb, rel_vb, _ = bufs[k % 2]

        def group(g, carry):
            tot = jnp.zeros((L,), jnp.float32)
            for j in range(L):
                row = g * L + j
                acc = None
                for c in range(DIM // L):
                    s = sub_vb[row, pl.ds(c * L, L)]
                    r = rel_vb[row, pl.ds(c * L, L)]
                    o = obj_vb[row, pl.ds(c * L, L)]
                    p = s * r * o
                    acc = p if acc is None else acc + p
                tot = jnp.where(col_idx == j, jnp.sum(acc), tot)
            out_v[pl.ds(k * CH + g * L, L)] = tot
            return carry

        lax.fori_loop(0, CH_GROUPS, group, 0)

    pending = fire(0)
    for k in range(N_CHUNKS):
        nxt = fire(k + 1) if k + 1 < N_CHUNKS else None
        for cp in pending:
            cp.wait()
        compute(k)
        pending = nxt

    pltpu.sync_copy(out_v, out_hbm.at[pl.ds(base, ROWS_PER_W)])


@functools.partial(
    pl.kernel,
    out_type=jax.ShapeDtypeStruct((BATCH,), jnp.float32),
    mesh=plsc.VectorSubcoreMesh(core_axis_name="c", subcore_axis_name="s"),
    compiler_params=pltpu.CompilerParams(needs_layout_passes=False),
    scratch_types=[
        pltpu.VMEM((ROWS_PER_W,), jnp.int32),
        pltpu.VMEM((CH, DIM), jnp.float32),
        pltpu.VMEM((CH, DIM), jnp.float32),
        pltpu.VMEM((CH, PAD_DIM), jnp.float32),
        pltpu.VMEM((CH, DIM), jnp.float32),
        pltpu.VMEM((CH, DIM), jnp.float32),
        pltpu.VMEM((CH, PAD_DIM), jnp.float32),
        pltpu.VMEM((ROWS_PER_W,), jnp.float32),
        pltpu.SemaphoreType.DMA,
        pltpu.SemaphoreType.DMA,
    ],
)
def _dist_mult_sc(sub_hbm, obj_hbm, rela_hbm, diag_hbm, out_hbm, *scratch):
    _sc_body(sub_hbm, obj_hbm, rela_hbm, diag_hbm, out_hbm, *scratch)


def kernel(sub_embed, obj_embed, rela, diag):
    diag_pad = jnp.pad(diag, ((0, 0), (0, PAD_DIM - DIM)))
    return _dist_mult_sc(sub_embed, obj_embed, rela.astype(jnp.int32), diag_pad)


# speedup vs baseline: 1.4655x; 1.1436x over previous
import functools

import jax
import jax.numpy as jnp
from jax import lax
from jax.experimental import pallas as pl
from jax.experimental.pallas import tpu as pltpu
from jax.experimental.pallas import tpu_sc as plsc

DIM = 64
NREL = 1000
BATCH = 16384
NC = 2
NS = 16
NW = NC * NS
COLS_PER_W = BATCH // NW
L = 16
CH = 128
N_CHUNKS = COLS_PER_W // CH
CH_GROUPS = CH // L
D_UNROLL = 4


def _sc_body(subT_hbm, objT_hbm, rela_hbm, diagT_hbm, out_hbm,
             idx_v, diag_v, sub_v0, obj_v0, sub_v1, obj_v1,
             out_v, dsem, sem0, sem1):
    wid = lax.axis_index("s") * NC + lax.axis_index("c")
    base = wid * COLS_PER_W

    bufs = ((sub_v0, obj_v0, sem0), (sub_v1, obj_v1, sem1))

    dcp = pltpu.async_copy(diagT_hbm, diag_v, dsem)
    pltpu.sync_copy(rela_hbm.at[pl.ds(base, COLS_PER_W)], idx_v)

    def fire(k):
        sub_vb, obj_vb, semb = bufs[k % 2]
        cbase = base + k * CH
        return (
            pltpu.async_copy(subT_hbm.at[:, pl.ds(cbase, CH)], sub_vb, semb),
            pltpu.async_copy(objT_hbm.at[:, pl.ds(cbase, CH)], obj_vb, semb),
        )

    def compute(k):
        sub_vb, obj_vb, _ = bufs[k % 2]
        idxs = [idx_v[pl.ds(k * CH + bg * L, L)] for bg in range(CH_GROUPS)]

        def dstep(i, accs):
            new = list(accs)
            for u in range(D_UNROLL):
                d = i * D_UNROLL + u
                dsplat = jnp.full((L,), d, jnp.int32)
                for bg in range(CH_GROUPS):
                    s = sub_vb[d, pl.ds(bg * L, L)]
                    o = obj_vb[d, pl.ds(bg * L, L)]
                    r = plsc.load_gather(diag_v, [dsplat, idxs[bg]])
                    new[bg] = new[bg] + s * o * r
            return tuple(new)

        zero = jnp.zeros((L,), jnp.float32)
        accs = lax.fori_loop(0, DIM // D_UNROLL, dstep,
                             tuple(zero for _ in range(CH_GROUPS)))
        for bg in range(CH_GROUPS):
            out_v[pl.ds(k * CH + bg * L, L)] = accs[bg]

    pending = fire(0)
    dcp.wait()
    for k in range(N_CHUNKS):
        nxt = fire(k + 1) if k + 1 < N_CHUNKS else None
        for cp in pending:
            cp.wait()
        compute(k)
        pending = nxt

    pltpu.sync_copy(out_v, out_hbm.at[pl.ds(base, COLS_PER_W)])


@functools.partial(
    pl.kernel,
    out_type=jax.ShapeDtypeStruct((BATCH,), jnp.float32),
    mesh=plsc.VectorSubcoreMesh(core_axis_name="c", subcore_axis_name="s"),
    compiler_params=pltpu.CompilerParams(needs_layout_passes=False),
    scratch_types=[
        pltpu.VMEM((COLS_PER_W,), jnp.int32),
        pltpu.VMEM((DIM, NREL), jnp.float32),
        pltpu.VMEM((DIM, CH), jnp.float32),
        pltpu.VMEM((DIM, CH), jnp.float32),
        pltpu.VMEM((DIM, CH), jnp.float32),
        pltpu.VMEM((DIM, CH), jnp.float32),
        pltpu.VMEM((COLS_PER_W,), jnp.float32),
        pltpu.SemaphoreType.DMA,
        pltpu.SemaphoreType.DMA,
        pltpu.SemaphoreType.DMA,
    ],
)
def _dist_mult_sc(subT_hbm, objT_hbm, rela_hbm, diagT_hbm, out_hbm, *scratch):
    _sc_body(subT_hbm, objT_hbm, rela_hbm, diagT_hbm, out_hbm, *scratch)


def kernel(sub_embed, obj_embed, rela, diag):
    return _dist_mult_sc(sub_embed.T, obj_embed.T, rela.astype(jnp.int32),
                         diag.T)


# speedup vs baseline: 1.8188x; 1.2411x over previous
import functools

import jax
import jax.numpy as jnp
from jax import lax
from jax.experimental import pallas as pl
from jax.experimental.pallas import tpu as pltpu
from jax.experimental.pallas import tpu_sc as plsc

DIM = 64
NREL = 1000
BATCH = 16384
NC = 2
NS = 16
NW = NC * NS
COLS_PER_W = BATCH // NW
L = 16
CH = 128
N_CHUNKS = COLS_PER_W // CH
CH_GROUPS = CH // L
D_UNROLL = 4


def _sc_body(subT_hbm, objT_hbm, rela_hbm, diagT_hbm, out_hbm,
             idx_v, diag_v, sub_v0, obj_v0, sub_v1, obj_v1,
             out_v, dsem, sem0, sem1):
    wid = lax.axis_index("s") * NC + lax.axis_index("c")
    base = wid * COLS_PER_W

    bufs = ((sub_v0, obj_v0, sem0), (sub_v1, obj_v1, sem1))

    dcp = pltpu.async_copy(diagT_hbm, diag_v, dsem)
    pltpu.sync_copy(rela_hbm.at[pl.ds(base, COLS_PER_W)], idx_v)

    def fire(k):
        sub_vb, obj_vb, semb = bufs[k % 2]
        cbase = base + k * CH
        return (
            pltpu.async_copy(subT_hbm.at[:, pl.ds(cbase, CH)], sub_vb, semb),
            pltpu.async_copy(objT_hbm.at[:, pl.ds(cbase, CH)], obj_vb, semb),
        )

    def compute(k):
        sub_vb, obj_vb, _ = bufs[k % 2]

        def bgroup(bg, carry):
            idx = idx_v[pl.ds(k * CH + bg * L, L)]
            acc = None
            for d in range(DIM):
                s = sub_vb[d, pl.ds(bg * L, L)]
                o = obj_vb[d, pl.ds(bg * L, L)]
                r = plsc.load_gather(
                    diag_v, [jnp.full((L,), d, jnp.int32), idx])
                p = s * o * r
                acc = p if acc is None else acc + p
            out_v[pl.ds(k * CH + bg * L, L)] = acc
            return carry

        lax.fori_loop(0, CH_GROUPS, bgroup, 0)

    pending = fire(0)
    dcp.wait()
    for k in range(N_CHUNKS):
        nxt = fire(k + 1) if k + 1 < N_CHUNKS else None
        for cp in pending:
            cp.wait()
        compute(k)
        pending = nxt

    pltpu.sync_copy(out_v, out_hbm.at[pl.ds(base, COLS_PER_W)])


@functools.partial(
    pl.kernel,
    out_type=jax.ShapeDtypeStruct((BATCH,), jnp.float32),
    mesh=plsc.VectorSubcoreMesh(core_axis_name="c", subcore_axis_name="s"),
    compiler_params=pltpu.CompilerParams(needs_layout_passes=False),
    scratch_types=[
        pltpu.VMEM((COLS_PER_W,), jnp.int32),
        pltpu.VMEM((DIM, NREL), jnp.float32),
        pltpu.VMEM((DIM, CH), jnp.float32),
        pltpu.VMEM((DIM, CH), jnp.float32),
        pltpu.VMEM((DIM, CH), jnp.float32),
        pltpu.VMEM((DIM, CH), jnp.float32),
        pltpu.VMEM((COLS_PER_W,), jnp.float32),
        pltpu.SemaphoreType.DMA,
        pltpu.SemaphoreType.DMA,
        pltpu.SemaphoreType.DMA,
    ],
)
def _dist_mult_sc(subT_hbm, objT_hbm, rela_hbm, diagT_hbm, out_hbm, *scratch):
    _sc_body(subT_hbm, objT_hbm, rela_hbm, diagT_hbm, out_hbm, *scratch)


def kernel(sub_embed, obj_embed, rela, diag):
    return _dist_mult_sc(sub_embed.T, obj_embed.T, rela.astype(jnp.int32),
                         diag.T)
